# baseline (speedup 1.0000x reference)
import functools

import jax
import jax.numpy as jnp
from jax import lax
from jax.experimental import pallas as pl
from jax.experimental.pallas import tpu as pltpu
from jax.experimental.pallas import tpu_sc as plsc

_NC, _NS, _L = 2, 16, 16
_NW = _NC * _NS

_B = 16384 * 100
_BPW = _B // _NW
_K = 12800
_CH = _BPW // _K

_mesh = plsc.VectorSubcoreMesh(core_axis_name="c", subcore_axis_name="s")


@functools.partial(
    pl.kernel,
    out_type=jax.ShapeDtypeStruct((2 * _B,), jnp.int32),
    mesh=_mesh,
    scratch_types=[
        pltpu.VMEM((2 * _K,), jnp.int32),
        pltpu.VMEM((_K,), jnp.int32),
        pltpu.VMEM((_K,), jnp.float32),
        pltpu.VMEM((2 * _K,), jnp.int32),
        pltpu.SemaphoreType.DMA,
    ],
    compiler_params=pltpu.CompilerParams(needs_layout_passes=False),
)
def _gather_mask(idx2_hbm, table_hbm, out_hbm, idx2_v, idx_v, vals_v, out2_v, sem):
    wid = lax.axis_index("s") * _NC + lax.axis_index("c")
    base = wid * jnp.int32(_BPW)

    zeros = jnp.zeros((_L,), jnp.int32)
    iota2 = lax.iota(jnp.int32, _L) * jnp.int32(2)
    two = jnp.int32(2)

    @pl.loop(jnp.int32(0), jnp.int32(2 * _K), step=jnp.int32(_L))
    def _(off2):
        out2_v[pl.ds(off2, _L)] = zeros

    @pl.loop(jnp.int32(0), jnp.int32(_CH))
    def _(ci):
        cbase = base + ci * jnp.int32(_K)
        pltpu.sync_copy(idx2_hbm.at[pl.ds(cbase * two, 2 * _K)], idx2_v)

        @pl.loop(jnp.int32(0), jnp.int32(_K), step=jnp.int32(_L))
        def _(off):
            lo = plsc.load_gather(idx2_v, [iota2 + off * two])
            idx_v[pl.ds(off, _L)] = lo

        pltpu.async_copy(table_hbm.at[idx_v], vals_v, sem).wait()

        @pl.loop(jnp.int32(0), jnp.int32(_K), step=jnp.int32(_L))
        def _(off):
            x = vals_v[pl.ds(off, _L)]
            y = jnp.int32(1) - x.astype(jnp.int32)
            plsc.store_scatter(out2_v, [iota2 + off * two], y)

        pltpu.sync_copy(out2_v, out_hbm.at[pl.ds(cbase * two, 2 * _K)])


def kernel(donors_idx, mask_fit_X_col):
    idx2 = lax.bitcast_convert_type(donors_idx.reshape(-1), jnp.int32).reshape(-1)
    out2 = _gather_mask(idx2, mask_fit_X_col)
    out = lax.bitcast_convert_type(out2.reshape(_B, 2), jnp.int64)
    return out.reshape(donors_idx.shape)


# speedup vs baseline: 10.2861x; 10.2861x over previous
import functools

import jax
import jax.numpy as jnp
from jax import lax
from jax.experimental import pallas as pl
from jax.experimental.pallas import tpu as pltpu
from jax.experimental.pallas import tpu_sc as plsc

_NC, _NS, _L = 2, 16, 16
_NW = _NC * _NS

_B = 16384 * 100
_BPW = _B // _NW

_mesh = plsc.VectorSubcoreMesh(core_axis_name="c", subcore_axis_name="s")


@functools.partial(
    pl.kernel,
    out_type=jax.ShapeDtypeStruct((_B,), jnp.int32),
    mesh=_mesh,
    scratch_types=[
        pltpu.VMEM((_BPW,), jnp.int32),
        pltpu.VMEM((_BPW,), jnp.float32),
        pltpu.SemaphoreType.DMA,
    ],
)
def _gather_mask(idx_hbm, table_hbm, out_hbm, idx_v, vals_v, sem):
    wid = lax.axis_index("s") * _NC + lax.axis_index("c")
    base = wid * jnp.int32(_BPW)
    pltpu.sync_copy(idx_hbm.at[pl.ds(base, _BPW)], idx_v)
    pltpu.async_copy(table_hbm.at[idx_v], vals_v, sem).wait()
    idx32_v = idx_v

    @pl.loop(jnp.int32(0), jnp.int32(_BPW), step=jnp.int32(_L))
    def _(off):
        sl = pl.ds(off, _L)
        idx32_v[sl] = jnp.int32(1) - vals_v[sl].astype(jnp.int32)

    pltpu.sync_copy(idx32_v, out_hbm.at[pl.ds(base, _BPW)])


def kernel(donors_idx, mask_fit_X_col):
    idx32 = donors_idx.astype(jnp.int32).reshape(-1)
    masked = _gather_mask(idx32, mask_fit_X_col)
    return masked.reshape(donors_idx.shape).astype(donors_idx.dtype)


# speedup vs baseline: 11.3844x; 1.1068x over previous
import functools

import jax
import jax.numpy as jnp
from jax import lax
from jax.experimental import pallas as pl
from jax.experimental.pallas import tpu as pltpu
from jax.experimental.pallas import tpu_sc as plsc

_NC, _NS, _L = 2, 16, 16
_NW = _NC * _NS

_R, _C = 16384, 100
_RPW = _R // _NW

_mesh = plsc.VectorSubcoreMesh(core_axis_name="c", subcore_axis_name="s")


@functools.partial(
    pl.kernel,
    out_type=jax.ShapeDtypeStruct((_R, _C), jnp.int32),
    mesh=_mesh,
    scratch_types=[
        pltpu.VMEM((_RPW, _C), jnp.int32),
        pltpu.VMEM((_RPW, _C), jnp.float32),
        pltpu.SemaphoreType.DMA,
    ],
    compiler_params=pltpu.CompilerParams(needs_layout_passes=False),
)
def _gather_mask(idx_hbm, table_hbm, out_hbm, idx_v, vals_v, sem):
    wid = lax.axis_index("s") * _NC + lax.axis_index("c")
    r0 = wid * jnp.int32(_RPW)

    pltpu.sync_copy(idx_hbm.at[pl.ds(r0, _RPW), :], idx_v)

    @pl.loop(jnp.int32(0), jnp.int32(_RPW))
    def _(r):
        pltpu.async_copy(table_hbm.at[idx_v.at[r]], vals_v.at[r], sem)

    @pl.loop(jnp.int32(0), jnp.int32(_RPW))
    def _(r):
        pltpu.make_async_copy(table_hbm.at[idx_v.at[r]], vals_v.at[r], sem).wait()

    @pl.loop(jnp.int32(0), jnp.int32(_RPW))
    def _(r):
        for c in (0, 16, 32, 48, 64, 80, _C - _L):
            sl = pl.ds(jnp.int32(c), _L)
            idx_v[r, sl] = jnp.int32(1) - vals_v[r, sl].astype(jnp.int32)

    pltpu.sync_copy(idx_v, out_hbm.at[pl.ds(r0, _RPW), :])


def kernel(donors_idx, mask_fit_X_col):
    idx32 = donors_idx.astype(jnp.int32)
    out = _gather_mask(idx32, mask_fit_X_col)
    return out.astype(donors_idx.dtype)
